# baseline (speedup 1.0000x reference)
import functools

import jax
import jax.numpy as jnp
import numpy as np
from jax import lax
from jax.experimental import pallas as pl
from jax.experimental.pallas import tpu as pltpu
from jax.experimental.pallas import tpu_sc as plsc

_B, _F, _M = 32, 64, 128
_N = _B * _F
_T = _M * (_M - 1) // 2
_L = 16
_G = 16
_KSC = 640

_R, _C = np.triu_indices(_M, k=1)
_ROWS = np.asarray(_R, dtype=np.int32)
_COLS = np.asarray(_C, dtype=np.int32)

_LENS = [_M - 1 - r for r in range(_M - 1)]
_OFF = np.concatenate([[0], np.cumsum(_LENS)]).astype(np.int64)


def _sc_call(x3, rows, cols):
    info = plsc.get_sparse_core_info()
    nw = info.num_cores * info.num_subcores
    npw = _KSC // nw

    mesh = plsc.VectorSubcoreMesh(core_axis_name="c", subcore_axis_name="s")

    @functools.partial(
        pl.kernel,
        mesh=mesh,
        out_type=jax.ShapeDtypeStruct((_KSC, _T), jnp.float32),
        scratch_types=[
            pltpu.VMEM((_M, _M), jnp.float32),
            pltpu.VMEM((_M, _M), jnp.float32),
            pltpu.VMEM((_T,), jnp.float32),
            pltpu.VMEM((_T,), jnp.float32),
            pltpu.VMEM((_T,), jnp.int32),
            pltpu.VMEM((_T,), jnp.int32),
            pltpu.SemaphoreType.DMA,
            pltpu.SemaphoreType.DMA,
        ],
        compiler_params=pltpu.CompilerParams(needs_layout_passes=False),
    )
    def k(x_hbm, r_hbm, c_hbm, out_hbm, in0, in1, ou0, ou1, rb, cb,
          in_sem, out_sem):
        wid = lax.axis_index("s") * info.num_cores + lax.axis_index("c")
        base = wid * npw
        pltpu.sync_copy(r_hbm, rb)
        pltpu.sync_copy(c_hbm, cb)
        ins = (in0, in1)
        ous = (ou0, ou1)

        def gather(sb, db):
            def chunk(i, c):
                riv = rb[pl.ds(i * _L, _L)]
                civ = cb[pl.ds(i * _L, _L)]
                db[pl.ds(i * _L, _L)] = plsc.load_gather(sb, [riv, civ])
                return c

            lax.fori_loop(0, _T // _L, chunk, 0, unroll=8)

        pltpu.async_copy(x_hbm.at[base], in0, in_sem)

        def outer(i, carry):
            n = base + 2 * i
            for b in range(2):
                m = n + b

                @pl.when(m + 1 < base + npw)
                def _():
                    pltpu.async_copy(x_hbm.at[m + 1], ins[1 - b], in_sem)

                pltpu.make_async_copy(x_hbm.at[m], ins[b], in_sem).wait()

                @pl.when(m - 2 >= base)
                def _():
                    pltpu.make_async_copy(
                        ous[b], out_hbm.at[m - 2], out_sem
                    ).wait()

                gather(ins[b], ous[b])
                pltpu.async_copy(ous[b], out_hbm.at[m], out_sem)
            return carry

        lax.fori_loop(0, npw // 2, outer, 0)
        pltpu.make_async_copy(ou0, out_hbm.at[base], out_sem).wait()
        pltpu.make_async_copy(ou1, out_hbm.at[base], out_sem).wait()

    return k(x3, rows, cols)


def _tc_body(x_ref, o_ref):
    for r in range(_M - 1):
        seg = _M - 1 - r
        o = int(_OFF[r])
        o_ref[:, o:o + seg] = x_ref[:, r, r + 1:_M]


def _tc_call(x3):
    n_tc = _N - _KSC
    off = _KSC // _G
    return pl.pallas_call(
        _tc_body,
        grid=(n_tc // _G,),
        in_specs=[
            pl.BlockSpec((_G, _M, _M), lambda i: (i + off, 0, 0)),
        ],
        out_specs=pl.BlockSpec((_G, _T), lambda i: (i + off, 0)),
        out_shape=jax.ShapeDtypeStruct((_N, _T), jnp.float32),
    )(x3)


@jax.jit
def _triu(x3, rows, cols):
    out_sc = _sc_call(x3, rows, cols)
    out_tc = _tc_call(x3)
    return lax.dynamic_update_slice(out_tc, out_sc, (0, 0))


def kernel(X):
    out = _triu(X.reshape(_N, _M, _M), _ROWS, _COLS)
    return out.reshape(_B, _F, _T)


# speedup vs baseline: 1.0923x; 1.0923x over previous
import functools

import jax
import jax.numpy as jnp
import numpy as np
from jax import lax
from jax.experimental import pallas as pl
from jax.experimental.pallas import tpu as pltpu
from jax.experimental.pallas import tpu_sc as plsc

_B, _F, _M = 32, 64, 128
_N = _B * _F
_T = _M * (_M - 1) // 2
_L = 16
_G = 16
_KSC = 576

_R, _C = np.triu_indices(_M, k=1)
_ROWS = np.asarray(_R, dtype=np.int32)
_COLS = np.asarray(_C, dtype=np.int32)

_LENS = [_M - 1 - r for r in range(_M - 1)]
_OFF = np.concatenate([[0], np.cumsum(_LENS)]).astype(np.int64)


def _sc_call(x3, rows, cols):
    info = plsc.get_sparse_core_info()
    nw = info.num_cores * info.num_subcores
    npw = _KSC // nw

    mesh = plsc.VectorSubcoreMesh(core_axis_name="c", subcore_axis_name="s")

    @functools.partial(
        pl.kernel,
        mesh=mesh,
        out_type=jax.ShapeDtypeStruct((_KSC, _T), jnp.float32),
        scratch_types=[
            pltpu.VMEM((_M, _M), jnp.float32),
            pltpu.VMEM((_M, _M), jnp.float32),
            pltpu.VMEM((_T,), jnp.float32),
            pltpu.VMEM((_T,), jnp.float32),
            pltpu.VMEM((_T,), jnp.int32),
            pltpu.VMEM((_T,), jnp.int32),
            pltpu.SemaphoreType.DMA,
            pltpu.SemaphoreType.DMA,
        ],
        compiler_params=pltpu.CompilerParams(needs_layout_passes=False),
    )
    def k(x_hbm, r_hbm, c_hbm, out_hbm, in0, in1, ou0, ou1, rb, cb,
          in_sem, out_sem):
        wid = lax.axis_index("s") * info.num_cores + lax.axis_index("c")
        base = wid * npw
        pltpu.sync_copy(r_hbm, rb)
        pltpu.sync_copy(c_hbm, cb)
        ins = (in0, in1)
        ous = (ou0, ou1)

        def gather(sb, db):
            def chunk(i, c):
                riv = rb[pl.ds(i * _L, _L)]
                civ = cb[pl.ds(i * _L, _L)]
                db[pl.ds(i * _L, _L)] = plsc.load_gather(sb, [riv, civ])
                return c

            lax.fori_loop(0, _T // _L, chunk, 0, unroll=8)

        pltpu.async_copy(x_hbm.at[base], in0, in_sem)

        def outer(i, carry):
            n = base + 2 * i
            for b in range(2):
                m = n + b

                @pl.when(m + 1 < base + npw)
                def _():
                    pltpu.async_copy(x_hbm.at[m + 1], ins[1 - b], in_sem)

                pltpu.make_async_copy(x_hbm.at[m], ins[b], in_sem).wait()

                @pl.when(m - 2 >= base)
                def _():
                    pltpu.make_async_copy(
                        ous[b], out_hbm.at[m - 2], out_sem
                    ).wait()

                gather(ins[b], ous[b])
                pltpu.async_copy(ous[b], out_hbm.at[m], out_sem)
            return carry

        lax.fori_loop(0, npw // 2, outer, 0)
        pltpu.make_async_copy(ou0, out_hbm.at[base], out_sem).wait()
        pltpu.make_async_copy(ou1, out_hbm.at[base], out_sem).wait()

    return k(x3, rows, cols)


def _tc_body(x_ref, o_ref):
    for r in range(_M - 1):
        seg = _M - 1 - r
        o = int(_OFF[r])
        o_ref[:, o:o + seg] = x_ref[:, r, r + 1:_M]


def _tc_call(x3):
    n_tc = _N - _KSC
    off = _KSC // _G
    return pl.pallas_call(
        _tc_body,
        grid=(n_tc // _G,),
        in_specs=[
            pl.BlockSpec((_G, _M, _M), lambda i: (i + off, 0, 0)),
        ],
        out_specs=pl.BlockSpec((_G, _T), lambda i: (i + off, 0)),
        out_shape=jax.ShapeDtypeStruct((_N, _T), jnp.float32),
    )(x3)


@jax.jit
def _triu(x3, rows, cols):
    out_sc = _sc_call(x3, rows, cols)
    out_tc = _tc_call(x3)
    return lax.dynamic_update_slice(out_tc, out_sc, (0, 0))


def kernel(X):
    out = _triu(X.reshape(_N, _M, _M), _ROWS, _COLS)
    return out.reshape(_B, _F, _T)
